# baseline (speedup 1.0000x reference)
import functools

import jax
import jax.numpy as jnp
from jax import lax
from jax.experimental import pallas as pl
from jax.experimental.pallas import tpu as pltpu
from jax.experimental.pallas import tpu_sc as plsc

B = 16384
EMB = 64
N_ROWS = 1000000
LINES = N_ROWS // 2
NC, NS = 2, 16
NW = NC * NS
BPW = B // NW
CHUNK = 128
NCHUNK = BPW // CHUNK

QCOLS = 512
QLINES = QCOLS // 2
NQUAD = 1953
TAIL_BASE = NQUAD * QCOLS * EMB


@functools.cache
def _get_sc_sweep():
    mesh = plsc.VectorSubcoreMesh(
        core_axis_name="c", subcore_axis_name="s", num_cores=NC, num_subcores=NS
    )

    @functools.partial(
        pl.kernel,
        mesh=mesh,
        out_type=jax.ShapeDtypeStruct((N_ROWS * EMB,), jnp.float32),
        scratch_types=[
            pltpu.VMEM((EMB, QCOLS), jnp.float32),
            pltpu.VMEM((QLINES * 2 * EMB,), jnp.float32),
            pltpu.VMEM((32 * 128,), jnp.float32),
        ],
        compiler_params=pltpu.CompilerParams(
            use_tc_tiling_on_sc=True, needs_layout_passes=False
        ),
    )
    def _sc_sweep(embT_hbm, tail_hbm, out_hbm, blk_v, ext_v, tail_v):
        wid = lax.axis_index("s") * NC + lax.axis_index("c")
        lane = lax.iota(jnp.int32, 16)
        jvecs = [j0 * 16 for j0 in range(EMB // 16)]

        def quad_body(t, _):
            q = wid + t * NW

            @pl.when(q < NQUAD)
            def _():
                pltpu.sync_copy(
                    embT_hbm.at[:, pl.ds(q * QCOLS, QCOLS)], blk_v
                )

                def line_body(n, _):
                    for p in range(2):
                        cvec = jnp.full((16,), 2 * n + p, jnp.int32)
                        for j0 in jvecs:
                            data = plsc.load_gather(
                                blk_v, [j0 + lane, cvec]
                            )
                            ext_v[pl.ds(n * 128 + p * EMB + j0, 16)] = data
                    return 0

                lax.fori_loop(0, QLINES, line_body, 0)
                pltpu.sync_copy(
                    ext_v,
                    out_hbm.at[pl.ds(q * QLINES * 2 * EMB, QLINES * 2 * EMB)],
                )

            return 0

        lax.fori_loop(0, (NQUAD + NW - 1) // NW, quad_body, 0)

        @pl.when(wid == 0)
        def _():
            pltpu.sync_copy(tail_hbm, tail_v)
            pltpu.sync_copy(tail_v, out_hbm.at[pl.ds(TAIL_BASE, 32 * 128)])

    return _sc_sweep


@functools.cache
def _get_sc_gather():
    mesh = plsc.VectorSubcoreMesh(
        core_axis_name="c", subcore_axis_name="s", num_cores=NC, num_subcores=NS
    )

    @functools.partial(
        pl.kernel,
        mesh=mesh,
        out_type=jax.ShapeDtypeStruct((3 * B, 2 * EMB), jnp.float32),
        scratch_types=[
            pltpu.VMEM((3 * BPW,), jnp.int32),
            pltpu.VMEM((BPW, 2 * EMB), jnp.float32),
            pltpu.SemaphoreType.DMA,
        ],
        compiler_params=pltpu.CompilerParams(use_tc_tiling_on_sc=True),
    )
    def _sc_gather(idx_hbm, tab_hbm, out_hbm, idx_v, rows_v, sem):
        wid = lax.axis_index("s") * NC + lax.axis_index("c")
        base = wid * BPW
        for l in range(3):
            pltpu.sync_copy(
                idx_hbm.at[pl.ds(l * B + base, BPW)],
                idx_v.at[pl.ds(l * BPW, BPW)],
            )
        for l in range(3):
            copies = []
            for j in range(NCHUNK):
                copies.append(
                    pltpu.async_copy(
                        tab_hbm.at[idx_v.at[pl.ds(l * BPW + j * CHUNK, CHUNK)]],
                        rows_v.at[pl.ds(j * CHUNK, CHUNK)],
                        sem,
                    )
                )
            for c in copies:
                c.wait()
            pltpu.sync_copy(rows_v, out_hbm.at[pl.ds(l * B + base, BPW)])

    return _sc_gather


RBLK = 2048
NBLK = B // RBLK


def _mlp_body(g_ref, par_ref, w1a_ref, w1b_ref, b1_ref, w2_ref, b2_ref,
              w3_ref, b3_ref, out_ref):
    def half(l):
        m = par_ref[l][:, None] > 0.5
        return jnp.where(m, g_ref[l, :, EMB:], g_ref[l, :, :EMB])

    avg = (half(0) + half(1)) * 0.5
    cc = half(2)
    h1 = jnp.dot(avg, w1a_ref[...], preferred_element_type=jnp.float32)
    h1 += jnp.dot(cc, w1b_ref[...], preferred_element_type=jnp.float32)
    h1 = jnp.maximum(h1 + b1_ref[...], 0.0)
    h2 = jnp.dot(h1, w2_ref[...], preferred_element_type=jnp.float32)
    h2 = jnp.maximum(h2 + b2_ref[...], 0.0)
    z = jnp.sum(h2 * w3_ref[...], axis=1) + b3_ref[0, 0]
    out_ref[...] = 1.0 / (1.0 + jnp.exp(-z))


_mlp = pl.pallas_call(
    _mlp_body,
    grid=(NBLK,),
    in_specs=[
        pl.BlockSpec((3, RBLK, 2 * EMB), lambda i: (0, i, 0)),
        pl.BlockSpec((3, RBLK), lambda i: (0, i)),
        pl.BlockSpec((EMB, 256), lambda i: (0, 0)),
        pl.BlockSpec((EMB, 256), lambda i: (0, 0)),
        pl.BlockSpec((1, 256), lambda i: (0, 0)),
        pl.BlockSpec((256, 128), lambda i: (0, 0)),
        pl.BlockSpec((1, 128), lambda i: (0, 0)),
        pl.BlockSpec((1, 128), lambda i: (0, 0)),
        pl.BlockSpec((1, 1), lambda i: (0, 0), memory_space=pltpu.SMEM),
    ],
    out_specs=pl.BlockSpec((RBLK,), lambda i: (i,)),
    out_shape=jax.ShapeDtypeStruct((B,), jnp.float32),
)


def kernel(drug_list1, drug_list2, cond_list, emb, W1, b1, W2, b2, W3, b3):
    embT = emb.T
    tail = jnp.concatenate(
        [emb[N_ROWS - 64 :: 2], emb[N_ROWS - 63 :: 2]], axis=1
    ).reshape(-1)
    tab = _get_sc_sweep()(embT, tail).reshape(LINES, 2 * EMB)
    idx = jnp.concatenate(
        [drug_list1, drug_list2, cond_list]
    ).astype(jnp.int32)
    g = _get_sc_gather()(idx >> 1, tab).reshape(3, B, 2 * EMB)
    par = (idx & 1).astype(jnp.float32).reshape(3, B)
    return _mlp(
        g,
        par,
        W1[:EMB],
        W1[EMB:],
        b1.reshape(1, -1),
        W2,
        b2.reshape(1, -1),
        W3.reshape(1, -1),
        b3.reshape(1, 1),
    )


# speedup vs baseline: 1.6534x; 1.6534x over previous
import functools

import jax
import jax.numpy as jnp
from jax import lax
from jax.experimental import pallas as pl
from jax.experimental.pallas import tpu as pltpu
from jax.experimental.pallas import tpu_sc as plsc

B = 16384
EMB = 64
N_ROWS = 1000000
LINES = N_ROWS // 2
NC, NS = 2, 16
NW = NC * NS
BPW = B // NW
CHUNK = 128
NCHUNK = BPW // CHUNK

QCOLS = 512
QLINES = QCOLS // 2
NQUAD = 1953
TAIL_BASE = NQUAD * QCOLS * EMB


@functools.cache
def _get_sc_sweep():
    mesh = plsc.VectorSubcoreMesh(
        core_axis_name="c", subcore_axis_name="s", num_cores=NC, num_subcores=NS
    )

    @functools.partial(
        pl.kernel,
        mesh=mesh,
        out_type=jax.ShapeDtypeStruct((N_ROWS * EMB,), jnp.float32),
        scratch_types=[
            pltpu.VMEM((EMB, QCOLS), jnp.float32),
            pltpu.VMEM((QLINES * 2 * EMB,), jnp.float32),
            pltpu.VMEM((32 * 128,), jnp.float32),
        ],
        compiler_params=pltpu.CompilerParams(
            use_tc_tiling_on_sc=True, needs_layout_passes=False
        ),
    )
    def _sc_sweep(embT_hbm, tail_hbm, out_hbm, blk_v, ext_v, tail_v):
        wid = lax.axis_index("s") * NC + lax.axis_index("c")
        lane = lax.iota(jnp.int32, 16)
        jvecs = [j0 * 16 for j0 in range(EMB // 16)]

        def quad_body(t, _):
            q = wid + t * NW

            @pl.when(q < NQUAD)
            def _():
                pltpu.sync_copy(
                    embT_hbm.at[:, pl.ds(q * QCOLS, QCOLS)], blk_v
                )

                @plsc.parallel_loop(0, QLINES, 1, unroll=8)
                def line_body(n):
                    for p in range(2):
                        cvec = jnp.full((16,), 2 * n + p, jnp.int32)
                        for j0 in jvecs:
                            data = plsc.load_gather(
                                blk_v, [j0 + lane, cvec]
                            )
                            ext_v[pl.ds(n * 128 + p * EMB + j0, 16)] = data
                pltpu.sync_copy(
                    ext_v,
                    out_hbm.at[pl.ds(q * QLINES * 2 * EMB, QLINES * 2 * EMB)],
                )

            return 0

        lax.fori_loop(0, (NQUAD + NW - 1) // NW, quad_body, 0)

        @pl.when(wid == 0)
        def _():
            pltpu.sync_copy(tail_hbm, tail_v)
            pltpu.sync_copy(tail_v, out_hbm.at[pl.ds(TAIL_BASE, 32 * 128)])

    return _sc_sweep


@functools.cache
def _get_sc_gather():
    mesh = plsc.VectorSubcoreMesh(
        core_axis_name="c", subcore_axis_name="s", num_cores=NC, num_subcores=NS
    )

    @functools.partial(
        pl.kernel,
        mesh=mesh,
        out_type=jax.ShapeDtypeStruct((3 * B, 2 * EMB), jnp.float32),
        scratch_types=[
            pltpu.VMEM((3 * BPW,), jnp.int32),
            pltpu.VMEM((BPW, 2 * EMB), jnp.float32),
            pltpu.SemaphoreType.DMA,
        ],
        compiler_params=pltpu.CompilerParams(use_tc_tiling_on_sc=True),
    )
    def _sc_gather(idx_hbm, tab_hbm, out_hbm, idx_v, rows_v, sem):
        wid = lax.axis_index("s") * NC + lax.axis_index("c")
        base = wid * BPW
        for l in range(3):
            pltpu.sync_copy(
                idx_hbm.at[pl.ds(l * B + base, BPW)],
                idx_v.at[pl.ds(l * BPW, BPW)],
            )
        for l in range(3):
            copies = []
            for j in range(NCHUNK):
                copies.append(
                    pltpu.async_copy(
                        tab_hbm.at[idx_v.at[pl.ds(l * BPW + j * CHUNK, CHUNK)]],
                        rows_v.at[pl.ds(j * CHUNK, CHUNK)],
                        sem,
                    )
                )
            for c in copies:
                c.wait()
            pltpu.sync_copy(rows_v, out_hbm.at[pl.ds(l * B + base, BPW)])

    return _sc_gather


RBLK = 2048
NBLK = B // RBLK


def _mlp_body(g_ref, par_ref, w1a_ref, w1b_ref, b1_ref, w2_ref, b2_ref,
              w3_ref, b3_ref, out_ref):
    def half(l):
        m = par_ref[l][:, None] > 0.5
        return jnp.where(m, g_ref[l, :, EMB:], g_ref[l, :, :EMB])

    avg = (half(0) + half(1)) * 0.5
    cc = half(2)
    h1 = jnp.dot(avg, w1a_ref[...], preferred_element_type=jnp.float32)
    h1 += jnp.dot(cc, w1b_ref[...], preferred_element_type=jnp.float32)
    h1 = jnp.maximum(h1 + b1_ref[...], 0.0)
    h2 = jnp.dot(h1, w2_ref[...], preferred_element_type=jnp.float32)
    h2 = jnp.maximum(h2 + b2_ref[...], 0.0)
    z = jnp.sum(h2 * w3_ref[...], axis=1) + b3_ref[0, 0]
    out_ref[...] = 1.0 / (1.0 + jnp.exp(-z))


_mlp = pl.pallas_call(
    _mlp_body,
    grid=(NBLK,),
    in_specs=[
        pl.BlockSpec((3, RBLK, 2 * EMB), lambda i: (0, i, 0)),
        pl.BlockSpec((3, RBLK), lambda i: (0, i)),
        pl.BlockSpec((EMB, 256), lambda i: (0, 0)),
        pl.BlockSpec((EMB, 256), lambda i: (0, 0)),
        pl.BlockSpec((1, 256), lambda i: (0, 0)),
        pl.BlockSpec((256, 128), lambda i: (0, 0)),
        pl.BlockSpec((1, 128), lambda i: (0, 0)),
        pl.BlockSpec((1, 128), lambda i: (0, 0)),
        pl.BlockSpec((1, 1), lambda i: (0, 0), memory_space=pltpu.SMEM),
    ],
    out_specs=pl.BlockSpec((RBLK,), lambda i: (i,)),
    out_shape=jax.ShapeDtypeStruct((B,), jnp.float32),
)


def kernel(drug_list1, drug_list2, cond_list, emb, W1, b1, W2, b2, W3, b3):
    embT = emb.T
    tail = jnp.concatenate(
        [emb[N_ROWS - 64 :: 2], emb[N_ROWS - 63 :: 2]], axis=1
    ).reshape(-1)
    tab = _get_sc_sweep()(embT, tail).reshape(LINES, 2 * EMB)
    idx = jnp.concatenate(
        [drug_list1, drug_list2, cond_list]
    ).astype(jnp.int32)
    g = _get_sc_gather()(idx >> 1, tab).reshape(3, B, 2 * EMB)
    par = (idx & 1).astype(jnp.float32).reshape(3, B)
    return _mlp(
        g,
        par,
        W1[:EMB],
        W1[EMB:],
        b1.reshape(1, -1),
        W2,
        b2.reshape(1, -1),
        W3.reshape(1, -1),
        b3.reshape(1, 1),
    )


# speedup vs baseline: 4.5288x; 2.7391x over previous
import functools

import jax
import jax.numpy as jnp
from jax import lax
from jax.experimental import pallas as pl
from jax.experimental.pallas import tpu as pltpu
from jax.experimental.pallas import tpu_sc as plsc

B = 16384
EMB = 64
N_ROWS = 1000000
LINES = N_ROWS // 2
NC, NS = 2, 16
NW = NC * NS
BPW = B // NW
CHUNK = 128
NCHUNK = BPW // CHUNK

TCOLS = 2048
NTBLK = (N_ROWS + TCOLS - 1) // TCOLS
SPLIT = (NTBLK // 2) * TCOLS
TLINES = SPLIT + (N_ROWS - 2 * SPLIT)


NHALF = NTBLK // 2


def _tr_body(xa_ref, xb_ref, out_ref):
    out_ref[:, :EMB] = xa_ref[...].T
    out_ref[:, EMB:] = xb_ref[...].T


_transpose = pl.pallas_call(
    _tr_body,
    grid=(NHALF + 1,),
    in_specs=[
        pl.BlockSpec((EMB, TCOLS), lambda i: (0, jnp.where(i < NHALF, i, 2 * NHALF))),
        pl.BlockSpec((EMB, TCOLS), lambda i: (0, jnp.where(i < NHALF, NHALF + i, 2 * NHALF))),
    ],
    out_specs=pl.BlockSpec((TCOLS, 2 * EMB), lambda i: (i, 0)),
    out_shape=jax.ShapeDtypeStruct((TLINES, 2 * EMB), jnp.float32),
)


@functools.cache
def _get_sc_gather():
    mesh = plsc.VectorSubcoreMesh(
        core_axis_name="c", subcore_axis_name="s", num_cores=NC, num_subcores=NS
    )

    @functools.partial(
        pl.kernel,
        mesh=mesh,
        out_type=jax.ShapeDtypeStruct((3 * B, 2 * EMB), jnp.float32),
        scratch_types=[
            pltpu.VMEM((3 * BPW,), jnp.int32),
            pltpu.VMEM((BPW, 2 * EMB), jnp.float32),
            pltpu.SemaphoreType.DMA,
        ],
        compiler_params=pltpu.CompilerParams(use_tc_tiling_on_sc=True),
    )
    def _sc_gather(idx_hbm, tab_hbm, out_hbm, idx_v, rows_v, sem):
        wid = lax.axis_index("s") * NC + lax.axis_index("c")
        base = wid * BPW
        for l in range(3):
            pltpu.sync_copy(
                idx_hbm.at[pl.ds(l * B + base, BPW)],
                idx_v.at[pl.ds(l * BPW, BPW)],
            )
        for l in range(3):
            copies = []
            for j in range(NCHUNK):
                copies.append(
                    pltpu.async_copy(
                        tab_hbm.at[idx_v.at[pl.ds(l * BPW + j * CHUNK, CHUNK)]],
                        rows_v.at[pl.ds(j * CHUNK, CHUNK)],
                        sem,
                    )
                )
            for c in copies:
                c.wait()
            pltpu.sync_copy(rows_v, out_hbm.at[pl.ds(l * B + base, BPW)])

    return _sc_gather


RBLK = 2048
NBLK = B // RBLK


def _mlp_body(g_ref, par_ref, w1a_ref, w1b_ref, b1_ref, w2_ref, b2_ref,
              w3_ref, b3_ref, out_ref):
    def half(l):
        m = par_ref[l][:, None] > 0.5
        return jnp.where(m, g_ref[l, :, EMB:], g_ref[l, :, :EMB])

    avg = (half(0) + half(1)) * 0.5
    cc = half(2)
    h1 = jnp.dot(avg, w1a_ref[...], preferred_element_type=jnp.float32)
    h1 += jnp.dot(cc, w1b_ref[...], preferred_element_type=jnp.float32)
    h1 = jnp.maximum(h1 + b1_ref[...], 0.0)
    h2 = jnp.dot(h1, w2_ref[...], preferred_element_type=jnp.float32)
    h2 = jnp.maximum(h2 + b2_ref[...], 0.0)
    z = jnp.sum(h2 * w3_ref[...], axis=1) + b3_ref[0, 0]
    out_ref[...] = 1.0 / (1.0 + jnp.exp(-z))


_mlp = pl.pallas_call(
    _mlp_body,
    grid=(NBLK,),
    in_specs=[
        pl.BlockSpec((3, RBLK, 2 * EMB), lambda i: (0, i, 0)),
        pl.BlockSpec((3, RBLK), lambda i: (0, i)),
        pl.BlockSpec((EMB, 256), lambda i: (0, 0)),
        pl.BlockSpec((EMB, 256), lambda i: (0, 0)),
        pl.BlockSpec((1, 256), lambda i: (0, 0)),
        pl.BlockSpec((256, 128), lambda i: (0, 0)),
        pl.BlockSpec((1, 128), lambda i: (0, 0)),
        pl.BlockSpec((1, 128), lambda i: (0, 0)),
        pl.BlockSpec((1, 1), lambda i: (0, 0), memory_space=pltpu.SMEM),
    ],
    out_specs=pl.BlockSpec((RBLK,), lambda i: (i,)),
    out_shape=jax.ShapeDtypeStruct((B,), jnp.float32),
)


def kernel(drug_list1, drug_list2, cond_list, emb, W1, b1, W2, b2, W3, b3):
    embT = emb.T
    tab = _transpose(embT, embT)
    idx = jnp.concatenate(
        [drug_list1, drug_list2, cond_list]
    ).astype(jnp.int32)
    line = jnp.where(idx < SPLIT, idx, idx - SPLIT)
    par = ((idx >= SPLIT) & (idx < 2 * SPLIT)).astype(jnp.float32)
    g = _get_sc_gather()(line, tab).reshape(3, B, 2 * EMB)
    par = par.reshape(3, B)
    return _mlp(
        g,
        par,
        W1[:EMB],
        W1[EMB:],
        b1.reshape(1, -1),
        W2,
        b2.reshape(1, -1),
        W3.reshape(1, -1),
        b3.reshape(1, 1),
    )


# speedup vs baseline: 5.4576x; 1.2051x over previous
import functools

import jax
import jax.numpy as jnp
from jax import lax
from jax.experimental import pallas as pl
from jax.experimental.pallas import tpu as pltpu
from jax.experimental.pallas import tpu_sc as plsc

B = 16384
EMB = 64
N_ROWS = 1000000
LINES = N_ROWS // 2
NC, NS = 2, 16
NW = NC * NS
BPW = B // NW
CHUNK = 128
NCHUNK = BPW // CHUNK

TCOLS = 4096
NTBLK = (N_ROWS + TCOLS - 1) // TCOLS
SPLIT = (NTBLK // 2) * TCOLS
TLINES = SPLIT + (N_ROWS - 2 * SPLIT)
NHALF = NTBLK // 2

_DOT_T = (((0,), (0,)), ((), ()))


def _tr_body(xa_ref, xb_ref, eye_ref, out_ref):
    e = eye_ref[...]
    out_ref[:, :EMB] = lax.dot_general(
        xa_ref[...], e, _DOT_T, preferred_element_type=jnp.float32
    )
    out_ref[:, EMB:] = lax.dot_general(
        xb_ref[...], e, _DOT_T, preferred_element_type=jnp.float32
    )


_transpose = pl.pallas_call(
    _tr_body,
    grid=(NHALF + 1,),
    in_specs=[
        pl.BlockSpec((EMB, TCOLS), lambda i: (0, jnp.where(i < NHALF, i, 2 * NHALF))),
        pl.BlockSpec((EMB, TCOLS), lambda i: (0, jnp.where(i < NHALF, NHALF + i, 2 * NHALF))),
        pl.BlockSpec((EMB, EMB), lambda i: (0, 0)),
    ],
    out_specs=pl.BlockSpec((TCOLS, 2 * EMB), lambda i: (i, 0)),
    out_shape=jax.ShapeDtypeStruct((TLINES, 2 * EMB), jnp.float32),
)


@functools.cache
def _get_sc_gather():
    mesh = plsc.VectorSubcoreMesh(
        core_axis_name="c", subcore_axis_name="s", num_cores=NC, num_subcores=NS
    )

    @functools.partial(
        pl.kernel,
        mesh=mesh,
        out_type=jax.ShapeDtypeStruct((3 * B, 2 * EMB), jnp.float32),
        scratch_types=[
            pltpu.VMEM((3 * BPW,), jnp.int32),
            pltpu.VMEM((BPW, 2 * EMB), jnp.float32),
            pltpu.SemaphoreType.DMA,
        ],
        compiler_params=pltpu.CompilerParams(use_tc_tiling_on_sc=True),
    )
    def _sc_gather(idx_hbm, tab_hbm, out_hbm, idx_v, rows_v, sem):
        wid = lax.axis_index("s") * NC + lax.axis_index("c")
        base = wid * BPW
        for l in range(3):
            pltpu.sync_copy(
                idx_hbm.at[pl.ds(l * B + base, BPW)],
                idx_v.at[pl.ds(l * BPW, BPW)],
            )
        for l in range(3):
            copies = []
            for j in range(NCHUNK):
                copies.append(
                    pltpu.async_copy(
                        tab_hbm.at[idx_v.at[pl.ds(l * BPW + j * CHUNK, CHUNK)]],
                        rows_v.at[pl.ds(j * CHUNK, CHUNK)],
                        sem,
                    )
                )
            for c in copies:
                c.wait()
            pltpu.sync_copy(rows_v, out_hbm.at[pl.ds(l * B + base, BPW)])

    return _sc_gather


RBLK = 2048
NBLK = B // RBLK


def _mlp_body(g_ref, par_ref, w1a_ref, w1b_ref, b1_ref, w2_ref, b2_ref,
              w3_ref, b3_ref, out_ref):
    def half(l):
        m = par_ref[l][:, None] > 0.5
        return jnp.where(m, g_ref[l, :, EMB:], g_ref[l, :, :EMB])

    avg = (half(0) + half(1)) * 0.5
    cc = half(2)
    h1 = jnp.dot(avg, w1a_ref[...], preferred_element_type=jnp.float32)
    h1 += jnp.dot(cc, w1b_ref[...], preferred_element_type=jnp.float32)
    h1 = jnp.maximum(h1 + b1_ref[...], 0.0)
    h2 = jnp.dot(h1, w2_ref[...], preferred_element_type=jnp.float32)
    h2 = jnp.maximum(h2 + b2_ref[...], 0.0)
    z = jnp.sum(h2 * w3_ref[...], axis=1) + b3_ref[0, 0]
    out_ref[...] = 1.0 / (1.0 + jnp.exp(-z))


_mlp = pl.pallas_call(
    _mlp_body,
    grid=(NBLK,),
    in_specs=[
        pl.BlockSpec((3, RBLK, 2 * EMB), lambda i: (0, i, 0)),
        pl.BlockSpec((3, RBLK), lambda i: (0, i)),
        pl.BlockSpec((EMB, 256), lambda i: (0, 0)),
        pl.BlockSpec((EMB, 256), lambda i: (0, 0)),
        pl.BlockSpec((1, 256), lambda i: (0, 0)),
        pl.BlockSpec((256, 128), lambda i: (0, 0)),
        pl.BlockSpec((1, 128), lambda i: (0, 0)),
        pl.BlockSpec((1, 128), lambda i: (0, 0)),
        pl.BlockSpec((1, 1), lambda i: (0, 0), memory_space=pltpu.SMEM),
    ],
    out_specs=pl.BlockSpec((RBLK,), lambda i: (i,)),
    out_shape=jax.ShapeDtypeStruct((B,), jnp.float32),
)


def kernel(drug_list1, drug_list2, cond_list, emb, W1, b1, W2, b2, W3, b3):
    embT = emb.T
    tab = _transpose(embT, embT, jnp.eye(EMB, dtype=jnp.float32))
    idx = jnp.concatenate(
        [drug_list1, drug_list2, cond_list]
    ).astype(jnp.int32)
    line = jnp.where(idx < SPLIT, idx, idx - SPLIT)
    par = ((idx >= SPLIT) & (idx < 2 * SPLIT)).astype(jnp.float32)
    g = _get_sc_gather()(line, tab).reshape(3, B, 2 * EMB)
    par = par.reshape(3, B)
    return _mlp(
        g,
        par,
        W1[:EMB],
        W1[EMB:],
        b1.reshape(1, -1),
        W2,
        b2.reshape(1, -1),
        W3.reshape(1, -1),
        b3.reshape(1, 1),
    )


# speedup vs baseline: 6.0762x; 1.1133x over previous
import functools

import jax
import jax.numpy as jnp
from jax import lax
from jax.experimental import pallas as pl
from jax.experimental.pallas import tpu as pltpu
from jax.experimental.pallas import tpu_sc as plsc

B = 16384
EMB = 64
N_ROWS = 1000000
LINES = N_ROWS // 2
NC, NS = 2, 16
NW = NC * NS
BPW = B // NW
CHUNK = 128
NCHUNK = BPW // CHUNK

TCOLS = 8192
NTBLK = (N_ROWS + TCOLS - 1) // TCOLS
SPLIT = (NTBLK // 2) * TCOLS
TLINES = SPLIT + (N_ROWS - 2 * SPLIT)
NHALF = NTBLK // 2

_DOT_T = (((0,), (0,)), ((), ()))


def _tr_body(xa_ref, xb_ref, eye_ref, out_ref):
    e = eye_ref[...]
    a = xa_ref[...].T
    b = lax.dot_general(
        xb_ref[...], e, _DOT_T, preferred_element_type=jnp.float32
    )
    out_ref[...] = jnp.concatenate([a, b], axis=1)


_transpose = pl.pallas_call(
    _tr_body,
    grid=(NHALF + 1,),
    compiler_params=pltpu.CompilerParams(
        dimension_semantics=("arbitrary",),
        fuse_transposed_lhs_in_matmul=True,
    ),
    in_specs=[
        pl.BlockSpec((EMB, TCOLS), lambda i: (0, jnp.where(i < NHALF, i, 2 * NHALF))),
        pl.BlockSpec((EMB, TCOLS), lambda i: (0, jnp.where(i < NHALF, NHALF + i, 2 * NHALF))),
        pl.BlockSpec((EMB, EMB), lambda i: (0, 0)),
    ],
    out_specs=pl.BlockSpec((TCOLS, 2 * EMB), lambda i: (i, 0)),
    out_shape=jax.ShapeDtypeStruct((TLINES, 2 * EMB), jnp.float32),
)


@functools.cache
def _get_sc_gather():
    mesh = plsc.VectorSubcoreMesh(
        core_axis_name="c", subcore_axis_name="s", num_cores=NC, num_subcores=NS
    )

    @functools.partial(
        pl.kernel,
        mesh=mesh,
        out_type=jax.ShapeDtypeStruct((3 * B, 2 * EMB), jnp.float32),
        scratch_types=[
            pltpu.VMEM((3 * BPW,), jnp.int32),
            pltpu.VMEM((BPW, 2 * EMB), jnp.float32),
            pltpu.SemaphoreType.DMA,
        ],
        compiler_params=pltpu.CompilerParams(use_tc_tiling_on_sc=True),
    )
    def _sc_gather(idx_hbm, tab_hbm, out_hbm, idx_v, rows_v, sem):
        wid = lax.axis_index("s") * NC + lax.axis_index("c")
        base = wid * BPW
        for l in range(3):
            pltpu.sync_copy(
                idx_hbm.at[pl.ds(l * B + base, BPW)],
                idx_v.at[pl.ds(l * BPW, BPW)],
            )
        for l in range(3):
            copies = []
            for j in range(NCHUNK):
                copies.append(
                    pltpu.async_copy(
                        tab_hbm.at[idx_v.at[pl.ds(l * BPW + j * CHUNK, CHUNK)]],
                        rows_v.at[pl.ds(j * CHUNK, CHUNK)],
                        sem,
                    )
                )
            for c in copies:
                c.wait()
            pltpu.sync_copy(rows_v, out_hbm.at[pl.ds(l * B + base, BPW)])

    return _sc_gather


RBLK = 2048
NBLK = B // RBLK


def _mlp_body(g_ref, par_ref, w1a_ref, w1b_ref, b1_ref, w2_ref, b2_ref,
              w3_ref, b3_ref, out_ref):
    def half(l):
        m = par_ref[l][:, None] > 0.5
        return jnp.where(m, g_ref[l, :, EMB:], g_ref[l, :, :EMB])

    avg = (half(0) + half(1)) * 0.5
    cc = half(2)
    h1 = jnp.dot(avg, w1a_ref[...], preferred_element_type=jnp.float32)
    h1 += jnp.dot(cc, w1b_ref[...], preferred_element_type=jnp.float32)
    h1 = jnp.maximum(h1 + b1_ref[...], 0.0)
    h2 = jnp.dot(h1, w2_ref[...], preferred_element_type=jnp.float32)
    h2 = jnp.maximum(h2 + b2_ref[...], 0.0)
    z = jnp.sum(h2 * w3_ref[...], axis=1) + b3_ref[0, 0]
    out_ref[...] = 1.0 / (1.0 + jnp.exp(-z))


_mlp = pl.pallas_call(
    _mlp_body,
    grid=(NBLK,),
    in_specs=[
        pl.BlockSpec((3, RBLK, 2 * EMB), lambda i: (0, i, 0)),
        pl.BlockSpec((3, RBLK), lambda i: (0, i)),
        pl.BlockSpec((EMB, 256), lambda i: (0, 0)),
        pl.BlockSpec((EMB, 256), lambda i: (0, 0)),
        pl.BlockSpec((1, 256), lambda i: (0, 0)),
        pl.BlockSpec((256, 128), lambda i: (0, 0)),
        pl.BlockSpec((1, 128), lambda i: (0, 0)),
        pl.BlockSpec((1, 128), lambda i: (0, 0)),
        pl.BlockSpec((1, 1), lambda i: (0, 0), memory_space=pltpu.SMEM),
    ],
    out_specs=pl.BlockSpec((RBLK,), lambda i: (i,)),
    out_shape=jax.ShapeDtypeStruct((B,), jnp.float32),
)


def kernel(drug_list1, drug_list2, cond_list, emb, W1, b1, W2, b2, W3, b3):
    embT = emb.T
    tab = _transpose(embT, embT, jnp.eye(EMB, dtype=jnp.float32))
    idx = jnp.concatenate(
        [drug_list1, drug_list2, cond_list]
    ).astype(jnp.int32)
    line = jnp.where(idx < SPLIT, idx, idx - SPLIT)
    par = ((idx >= SPLIT) & (idx < 2 * SPLIT)).astype(jnp.float32)
    g = _get_sc_gather()(line, tab).reshape(3, B, 2 * EMB)
    par = par.reshape(3, B)
    return _mlp(
        g,
        par,
        W1[:EMB],
        W1[EMB:],
        b1.reshape(1, -1),
        W2,
        b2.reshape(1, -1),
        W3.reshape(1, -1),
        b3.reshape(1, 1),
    )


# speedup vs baseline: 6.3636x; 1.0473x over previous
import functools

import jax
import jax.numpy as jnp
from jax import lax
from jax.experimental import pallas as pl
from jax.experimental.pallas import tpu as pltpu
from jax.experimental.pallas import tpu_sc as plsc

B = 16384
EMB = 64
N_ROWS = 1000000
LINES = N_ROWS // 2
NC, NS = 2, 16
NW = NC * NS
BPW = B // NW
CHUNK = 128
NCHUNK = BPW // CHUNK

TCOLS = 16384
NTBLK = (N_ROWS + TCOLS - 1) // TCOLS
NHALF = (NTBLK + 1) // 2
SPLIT = NHALF * TCOLS
TLINES = SPLIT

_DOT_T = (((0,), (0,)), ((), ()))


def _tr_body(xa_ref, xb_ref, eye_ref, out_ref):
    e = eye_ref[...]
    a = xa_ref[...].T
    b = lax.dot_general(
        xb_ref[...], e, _DOT_T, preferred_element_type=jnp.float32
    )
    out_ref[...] = jnp.concatenate([a, b], axis=1)


_transpose = pl.pallas_call(
    _tr_body,
    grid=(NHALF,),
    compiler_params=pltpu.CompilerParams(
        dimension_semantics=("arbitrary",),
        fuse_transposed_lhs_in_matmul=True,
    ),
    in_specs=[
        pl.BlockSpec((EMB, TCOLS), lambda i: (0, i)),
        pl.BlockSpec((EMB, TCOLS), lambda i: (0, NHALF + i)),
        pl.BlockSpec((EMB, EMB), lambda i: (0, 0)),
    ],
    out_specs=pl.BlockSpec((TCOLS, 2 * EMB), lambda i: (i, 0)),
    out_shape=jax.ShapeDtypeStruct((TLINES, 2 * EMB), jnp.float32),
)


@functools.cache
def _get_sc_gather():
    mesh = plsc.VectorSubcoreMesh(
        core_axis_name="c", subcore_axis_name="s", num_cores=NC, num_subcores=NS
    )

    @functools.partial(
        pl.kernel,
        mesh=mesh,
        out_type=jax.ShapeDtypeStruct((3 * B, 2 * EMB), jnp.float32),
        scratch_types=[
            pltpu.VMEM((3 * BPW,), jnp.int32),
            pltpu.VMEM((BPW, 2 * EMB), jnp.float32),
            pltpu.SemaphoreType.DMA,
        ],
        compiler_params=pltpu.CompilerParams(use_tc_tiling_on_sc=True),
    )
    def _sc_gather(idx_hbm, tab_hbm, out_hbm, idx_v, rows_v, sem):
        wid = lax.axis_index("s") * NC + lax.axis_index("c")
        base = wid * BPW
        for l in range(3):
            pltpu.sync_copy(
                idx_hbm.at[pl.ds(l * B + base, BPW)],
                idx_v.at[pl.ds(l * BPW, BPW)],
            )
        for l in range(3):
            copies = []
            for j in range(NCHUNK):
                copies.append(
                    pltpu.async_copy(
                        tab_hbm.at[idx_v.at[pl.ds(l * BPW + j * CHUNK, CHUNK)]],
                        rows_v.at[pl.ds(j * CHUNK, CHUNK)],
                        sem,
                    )
                )
            for c in copies:
                c.wait()
            pltpu.sync_copy(rows_v, out_hbm.at[pl.ds(l * B + base, BPW)])

    return _sc_gather


RBLK = 2048
NBLK = B // RBLK


def _mlp_body(g_ref, par_ref, w1a_ref, w1b_ref, b1_ref, w2_ref, b2_ref,
              w3_ref, b3_ref, out_ref):
    def half(l):
        m = par_ref[l][:, None] > 0.5
        return jnp.where(m, g_ref[l, :, EMB:], g_ref[l, :, :EMB])

    avg = (half(0) + half(1)) * 0.5
    cc = half(2)
    h1 = jnp.dot(avg, w1a_ref[...], preferred_element_type=jnp.float32)
    h1 += jnp.dot(cc, w1b_ref[...], preferred_element_type=jnp.float32)
    h1 = jnp.maximum(h1 + b1_ref[...], 0.0)
    h2 = jnp.dot(h1, w2_ref[...], preferred_element_type=jnp.float32)
    h2 = jnp.maximum(h2 + b2_ref[...], 0.0)
    z = jnp.sum(h2 * w3_ref[...], axis=1) + b3_ref[0, 0]
    out_ref[...] = 1.0 / (1.0 + jnp.exp(-z))


_mlp = pl.pallas_call(
    _mlp_body,
    grid=(NBLK,),
    in_specs=[
        pl.BlockSpec((3, RBLK, 2 * EMB), lambda i: (0, i, 0)),
        pl.BlockSpec((3, RBLK), lambda i: (0, i)),
        pl.BlockSpec((EMB, 256), lambda i: (0, 0)),
        pl.BlockSpec((EMB, 256), lambda i: (0, 0)),
        pl.BlockSpec((1, 256), lambda i: (0, 0)),
        pl.BlockSpec((256, 128), lambda i: (0, 0)),
        pl.BlockSpec((1, 128), lambda i: (0, 0)),
        pl.BlockSpec((1, 128), lambda i: (0, 0)),
        pl.BlockSpec((1, 1), lambda i: (0, 0), memory_space=pltpu.SMEM),
    ],
    out_specs=pl.BlockSpec((RBLK,), lambda i: (i,)),
    out_shape=jax.ShapeDtypeStruct((B,), jnp.float32),
)


def kernel(drug_list1, drug_list2, cond_list, emb, W1, b1, W2, b2, W3, b3):
    embT = emb.T
    tab = _transpose(embT, embT, jnp.eye(EMB, dtype=jnp.float32))
    idx = jnp.concatenate(
        [drug_list1, drug_list2, cond_list]
    ).astype(jnp.int32)
    line = jnp.where(idx < SPLIT, idx, idx - SPLIT)
    par = (idx >= SPLIT).astype(jnp.float32)
    g = _get_sc_gather()(line, tab).reshape(3, B, 2 * EMB)
    par = par.reshape(3, B)
    return _mlp(
        g,
        par,
        W1[:EMB],
        W1[EMB:],
        b1.reshape(1, -1),
        W2,
        b2.reshape(1, -1),
        W3.reshape(1, -1),
        b3.reshape(1, 1),
    )
